# Initial kernel scaffold; baseline (speedup 1.0000x reference)
import jax
import jax.numpy as jnp
from jax.experimental import pallas as pl


def kernel(x, batch, W, b):
    raise NotImplementedError("write your pallas kernel here")


# speedup vs baseline: 6.3380x; 6.3380x over previous
import functools

import jax
import jax.numpy as jnp
from jax import lax
from jax.experimental import pallas as pl
from jax.experimental.pallas import tpu as pltpu

N = 100000
D = 512
G = 512
BLK = 800
NBLK = N // BLK


def _pass_a(x_ref, batch_ref, w_ref, b_ref, ctx_ref, sums_ref, counts_ref):
    g = pl.program_id(0)

    @pl.when(g == 0)
    def _init():
        sums_ref[...] = jnp.zeros_like(sums_ref)
        counts_ref[...] = jnp.zeros_like(counts_ref)

    batch = batch_ref[0, 0, :]
    iota = lax.broadcasted_iota(jnp.int32, (BLK, G), 1)
    oh = (batch[:, None] == iota).astype(jnp.float32)
    x = x_ref[...]
    sums_ref[...] += lax.dot_general(
        oh, x, (((0,), (0,)), ((), ())), preferred_element_type=jnp.float32)
    counts_ref[...] += jnp.sum(oh, axis=0, keepdims=True)

    @pl.when(g == NBLK - 1)
    def _fin():
        counts = jnp.maximum(counts_ref[0, :], 1.0)
        mean = sums_ref[...] / counts[:, None]
        ctx = lax.dot_general(
            mean, w_ref[...], (((1,), (1,)), ((), ())),
            preferred_element_type=jnp.float32)
        ctx_ref[...] = jnp.tanh(ctx + b_ref[0, :][None, :])


def _pass_b(x_ref, batch_ref, ctx_ref, out_ref, acc_ref):
    g = pl.program_id(0)

    @pl.when(g == 0)
    def _init():
        acc_ref[...] = jnp.zeros_like(acc_ref)

    batch = batch_ref[0, 0, :]
    iota = lax.broadcasted_iota(jnp.int32, (BLK, G), 1)
    oh = (batch[:, None] == iota).astype(jnp.float32)
    x = x_ref[...]
    ctxn = lax.dot_general(
        oh, ctx_ref[...], (((1,), (0,)), ((), ())),
        preferred_element_type=jnp.float32)
    score = jax.nn.sigmoid(jnp.sum(x * ctxn, axis=1, keepdims=True))
    acc_ref[...] += lax.dot_general(
        oh, score * x, (((0,), (0,)), ((), ())),
        preferred_element_type=jnp.float32)

    @pl.when(g == NBLK - 1)
    def _fin():
        out_ref[...] = acc_ref[...]


@jax.jit
def kernel(x, batch, W, b):
    batch3 = batch.astype(jnp.int32).reshape(NBLK, 1, BLK)
    ctx = pl.pallas_call(
        _pass_a,
        grid=(NBLK,),
        in_specs=[
            pl.BlockSpec((BLK, D), lambda g: (g, 0)),
            pl.BlockSpec((1, 1, BLK), lambda g: (g, 0, 0)),
            pl.BlockSpec((D, D), lambda g: (0, 0)),
            pl.BlockSpec((1, D), lambda g: (0, 0)),
        ],
        out_specs=pl.BlockSpec((G, D), lambda g: (0, 0)),
        out_shape=jax.ShapeDtypeStruct((G, D), jnp.float32),
        scratch_shapes=[
            pltpu.VMEM((G, D), jnp.float32),
            pltpu.VMEM((1, G), jnp.float32),
        ],
    )(x, batch3, W, b.reshape(1, D))
    out = pl.pallas_call(
        _pass_b,
        grid=(NBLK,),
        in_specs=[
            pl.BlockSpec((BLK, D), lambda g: (g, 0)),
            pl.BlockSpec((1, 1, BLK), lambda g: (g, 0, 0)),
            pl.BlockSpec((G, D), lambda g: (0, 0)),
        ],
        out_specs=pl.BlockSpec((G, D), lambda g: (0, 0)),
        out_shape=jax.ShapeDtypeStruct((G, D), jnp.float32),
        scratch_shapes=[pltpu.VMEM((G, D), jnp.float32)],
    )(x, batch3, ctx)
    return out
